# Initial kernel scaffold; baseline (speedup 1.0000x reference)
import jax
import jax.numpy as jnp
from jax.experimental import pallas as pl


def kernel(indices, table):
    raise NotImplementedError("write your pallas kernel here")


# speedup vs baseline: 1.8984x; 1.8984x over previous
import functools

import jax
import jax.numpy as jnp
from jax import lax
from jax.experimental import pallas as pl
from jax.experimental.pallas import tpu as pltpu
from jax.experimental.pallas import tpu_sc as plsc

B = 16384
V = 17
D = 10
LANES = 16


def kernel(indices, table):
    info = plsc.get_sparse_core_info()
    num_workers = info.num_cores * info.num_subcores
    b_per_w = B // num_workers
    groups = b_per_w // LANES

    idx2 = indices.astype(jnp.int32).reshape(num_workers, b_per_w)
    table_flat = table.reshape(V * D)

    mesh = plsc.VectorSubcoreMesh(core_axis_name="c", subcore_axis_name="s")

    @functools.partial(
        pl.kernel,
        mesh=mesh,
        out_type=jax.ShapeDtypeStruct((B * D,), jnp.float32),
        scratch_types=[
            pltpu.VMEM((b_per_w,), jnp.int32),
            pltpu.VMEM((V * D,), jnp.float32),
            pltpu.VMEM((b_per_w * D,), jnp.float32),
            pltpu.SemaphoreType.DMA,
        ],
        compiler_params=pltpu.CompilerParams(needs_layout_passes=False),
    )
    def emb(idx_hbm, table_hbm, out_hbm, idx_v, table_v, rows_v, sem):
        wid = lax.axis_index("s") * info.num_cores + lax.axis_index("c")
        tbl_cp = pltpu.async_copy(table_hbm, table_v, sem)
        pltpu.sync_copy(idx_hbm.at[wid], idx_v)
        tbl_cp.wait()

        lane = lax.broadcasted_iota(jnp.int32, (LANES,), 0)
        lane10 = lane * D

        def body(g, _):
            idx16 = idx_v[pl.ds(g * LANES, LANES)]
            rowbase = idx16 * D
            off = lane10 + g * (LANES * D)
            for d in range(D):
                val = plsc.load_gather(table_v, [rowbase + d])
                plsc.store_scatter(rows_v, [off + d], val)
            return 0

        lax.fori_loop(0, groups, body, 0)
        pltpu.sync_copy(rows_v, out_hbm.at[pl.ds(wid * (b_per_w * D), b_per_w * D)])

    return emb(idx2, table_flat).reshape(B, D)


# speedup vs baseline: 2.1732x; 1.1447x over previous
import functools

import jax
import jax.numpy as jnp
from jax import lax
from jax.experimental import pallas as pl
from jax.experimental.pallas import tpu as pltpu
from jax.experimental.pallas import tpu_sc as plsc

B = 16384
V = 17
D = 10
LANES = 16


def kernel(indices, table):
    info = plsc.get_sparse_core_info()
    num_workers = info.num_cores * info.num_subcores
    b_per_w = B // num_workers
    groups = b_per_w // LANES

    mesh = plsc.VectorSubcoreMesh(core_axis_name="c", subcore_axis_name="s")

    @functools.partial(
        pl.kernel,
        mesh=mesh,
        out_type=jax.ShapeDtypeStruct((B, D), jnp.float32),
        scratch_types=[
            pltpu.VMEM((b_per_w,), jnp.int32),
            pltpu.VMEM((V, D), jnp.float32),
            pltpu.VMEM((b_per_w, D), jnp.float32),
            pltpu.SemaphoreType.DMA,
        ],
        compiler_params=pltpu.CompilerParams(needs_layout_passes=False),
    )
    def emb(idx_hbm, table_hbm, out_hbm, idx_v, table_v, rows_v, sem):
        wid = lax.axis_index("s") * info.num_cores + lax.axis_index("c")
        base = wid * b_per_w
        tbl_cp = pltpu.async_copy(table_hbm, table_v, sem)
        pltpu.sync_copy(idx_hbm.at[pl.ds(base, b_per_w)], idx_v)
        tbl_cp.wait()

        lane = lax.broadcasted_iota(jnp.int32, (LANES,), 0)

        def body(g, _):
            idx16 = idx_v[pl.ds(g * LANES, LANES)]
            brow = g * LANES + lane
            for d in range(D):
                dvec = jnp.full((LANES,), d, dtype=jnp.int32)
                val = plsc.load_gather(table_v, [idx16, dvec])
                plsc.store_scatter(rows_v, [brow, dvec], val)
            return 0

        lax.fori_loop(0, groups, body, 0)
        pltpu.sync_copy(rows_v, out_hbm.at[pl.ds(base, b_per_w)])

    return emb(indices.astype(jnp.int32), table)


# speedup vs baseline: 3.4204x; 1.5739x over previous
import functools

import jax
import jax.numpy as jnp
from jax import lax
from jax.experimental import pallas as pl
from jax.experimental.pallas import tpu as pltpu
from jax.experimental.pallas import tpu_sc as plsc

B = 16384
V = 17
D = 10
LANES = 16


def kernel(indices, table):
    info = plsc.get_sparse_core_info()
    num_workers = info.num_cores * info.num_subcores
    b_per_w = B // num_workers
    groups = b_per_w // LANES

    mesh = plsc.VectorSubcoreMesh(core_axis_name="c", subcore_axis_name="s")

    @functools.partial(
        pl.kernel,
        mesh=mesh,
        out_type=jax.ShapeDtypeStruct((D, B), jnp.float32),
        scratch_types=[
            pltpu.VMEM((b_per_w,), jnp.int32),
            pltpu.VMEM((D, V), jnp.float32),
            pltpu.VMEM((D, b_per_w), jnp.float32),
            pltpu.SemaphoreType.DMA,
        ],
        compiler_params=pltpu.CompilerParams(needs_layout_passes=False),
    )
    def emb(idx_hbm, table_hbm, out_hbm, idx_v, table_v, rows_v, sem):
        wid = lax.axis_index("s") * info.num_cores + lax.axis_index("c")
        base = wid * b_per_w
        tbl_cp = pltpu.async_copy(table_hbm, table_v, sem)
        pltpu.sync_copy(idx_hbm.at[pl.ds(base, b_per_w)], idx_v)
        tbl_cp.wait()

        def body(g, _):
            idx16 = idx_v[pl.ds(g * LANES, LANES)]
            for d in range(D):
                dvec = jnp.full((LANES,), d, dtype=jnp.int32)
                val = plsc.load_gather(table_v, [dvec, idx16])
                rows_v[d, pl.ds(g * LANES, LANES)] = val
            return 0

        lax.fori_loop(0, groups, body, 0)
        pltpu.sync_copy(rows_v, out_hbm.at[:, pl.ds(base, b_per_w)])

    return emb(indices.astype(jnp.int32), table.T).T


# speedup vs baseline: 3.4537x; 1.0097x over previous
import functools

import jax
import jax.numpy as jnp
from jax import lax
from jax.experimental import pallas as pl
from jax.experimental.pallas import tpu as pltpu
from jax.experimental.pallas import tpu_sc as plsc

B = 16384
V = 17
D = 10
LANES = 16


def kernel(indices, table):
    info = plsc.get_sparse_core_info()
    num_workers = info.num_cores * info.num_subcores
    b_per_w = B // num_workers
    groups = b_per_w // LANES

    mesh = plsc.VectorSubcoreMesh(core_axis_name="c", subcore_axis_name="s")

    n_chunks = 4
    cg = groups // n_chunks
    cols = cg * LANES

    @functools.partial(
        pl.kernel,
        mesh=mesh,
        out_type=jax.ShapeDtypeStruct((D, B), jnp.float32),
        scratch_types=[
            pltpu.VMEM((b_per_w,), jnp.int32),
            pltpu.VMEM((D, V), jnp.float32),
            pltpu.VMEM((D, b_per_w), jnp.float32),
            pltpu.SemaphoreType.DMA,
        ]
        + [pltpu.SemaphoreType.DMA] * n_chunks,
        compiler_params=pltpu.CompilerParams(needs_layout_passes=False),
    )
    def emb(idx_hbm, table_hbm, out_hbm, idx_v, table_v, rows_v, sem, *sems):
        wid = lax.axis_index("s") * info.num_cores + lax.axis_index("c")
        base = wid * b_per_w
        tbl_cp = pltpu.async_copy(table_hbm, table_v, sem)
        pltpu.sync_copy(idx_hbm.at[pl.ds(base, b_per_w)], idx_v)
        tbl_cp.wait()

        copies = []
        for c in range(n_chunks):
            @plsc.parallel_loop(0, cg, unroll=4)
            def body(g, c=c):
                gg = c * cg + g
                idx16 = idx_v[pl.ds(gg * LANES, LANES)]
                for d in range(D):
                    dvec = jnp.full((LANES,), d, dtype=jnp.int32)
                    val = plsc.load_gather(table_v, [dvec, idx16])
                    rows_v[d, pl.ds(gg * LANES, LANES)] = val
            copies.append(
                pltpu.async_copy(
                    rows_v.at[:, pl.ds(c * cols, cols)],
                    out_hbm.at[:, pl.ds(base + c * cols, cols)],
                    sems[c],
                )
            )
        for cp in copies:
            cp.wait()

    return emb(indices.astype(jnp.int32), table.T).T


# speedup vs baseline: 3.4975x; 1.0127x over previous
import functools

import jax
import jax.numpy as jnp
from jax import lax
from jax.experimental import pallas as pl
from jax.experimental.pallas import tpu as pltpu
from jax.experimental.pallas import tpu_sc as plsc

B = 16384
V = 17
D = 10
LANES = 16


def kernel(indices, table):
    info = plsc.get_sparse_core_info()
    num_workers = info.num_cores * info.num_subcores
    b_per_w = B // num_workers
    groups = b_per_w // LANES

    mesh = plsc.VectorSubcoreMesh(core_axis_name="c", subcore_axis_name="s")

    @functools.partial(
        pl.kernel,
        mesh=mesh,
        out_type=jax.ShapeDtypeStruct((D, B), jnp.float32),
        scratch_types=[
            pltpu.VMEM((b_per_w,), jnp.int32),
            pltpu.VMEM((D, V), jnp.float32),
            pltpu.VMEM((D, b_per_w), jnp.float32),
            pltpu.SemaphoreType.DMA,
        ],
        compiler_params=pltpu.CompilerParams(needs_layout_passes=False),
    )
    def emb(idx_hbm, table_hbm, out_hbm, idx_v, table_v, rows_v, sem):
        wid = lax.axis_index("s") * info.num_cores + lax.axis_index("c")
        base = wid * b_per_w
        tbl_cp = pltpu.async_copy(table_hbm, table_v, sem)
        pltpu.sync_copy(idx_hbm.at[pl.ds(base, b_per_w)], idx_v)
        tbl_cp.wait()

        @plsc.parallel_loop(0, groups, unroll=8)
        def body(g):
            idx16 = idx_v[pl.ds(g * LANES, LANES)]
            for d in range(D):
                dvec = jnp.full((LANES,), d, dtype=jnp.int32)
                val = plsc.load_gather(table_v, [dvec, idx16])
                rows_v[d, pl.ds(g * LANES, LANES)] = val
        pltpu.sync_copy(rows_v, out_hbm.at[:, pl.ds(base, b_per_w)])

    return emb(indices.astype(jnp.int32), table.T).T


# speedup vs baseline: 3.5276x; 1.0086x over previous
import functools

import jax
import jax.numpy as jnp
from jax import lax
from jax.experimental import pallas as pl
from jax.experimental.pallas import tpu as pltpu
from jax.experimental.pallas import tpu_sc as plsc

B = 16384
V = 17
D = 10
LANES = 16


def kernel(indices, table):
    info = plsc.get_sparse_core_info()
    num_workers = info.num_cores * info.num_subcores
    b_per_w = B // num_workers
    groups = b_per_w // LANES

    mesh = plsc.VectorSubcoreMesh(core_axis_name="c", subcore_axis_name="s")

    @functools.partial(
        pl.kernel,
        mesh=mesh,
        out_type=jax.ShapeDtypeStruct((D, B), jnp.float32),
        scratch_types=[
            pltpu.VMEM((b_per_w,), jnp.int32),
            pltpu.VMEM((D, V), jnp.float32),
            pltpu.VMEM((D, b_per_w), jnp.float32),
            pltpu.SemaphoreType.DMA,
        ],
        compiler_params=pltpu.CompilerParams(needs_layout_passes=False),
    )
    def emb(idx_hbm, table_hbm, out_hbm, idx_v, table_v, rows_v, sem):
        wid = lax.axis_index("s") * info.num_cores + lax.axis_index("c")
        base = wid * b_per_w
        tbl_cp = pltpu.async_copy(table_hbm, table_v, sem)
        pltpu.sync_copy(idx_hbm.at[pl.ds(base, b_per_w)], idx_v)
        tbl_cp.wait()

        @plsc.parallel_loop(0, groups, unroll=4)
        def body(g):
            idx16 = idx_v[pl.ds(g * LANES, LANES)]
            for d in range(D):
                dvec = jnp.full((LANES,), d, dtype=jnp.int32)
                val = plsc.load_gather(table_v, [dvec, idx16])
                rows_v[d, pl.ds(g * LANES, LANES)] = val
        pltpu.sync_copy(rows_v, out_hbm.at[:, pl.ds(base, b_per_w)])

    return emb(indices.astype(jnp.int32), table.T).T
